# Initial kernel scaffold; baseline (speedup 1.0000x reference)
import jax
import jax.numpy as jnp
from jax.experimental import pallas as pl


def kernel(x, edge_index, params):
    raise NotImplementedError("write your pallas kernel here")


# speedup vs baseline: 6.3773x; 6.3773x over previous
import functools

import jax
import jax.numpy as jnp
from jax import lax
from jax.experimental import pallas as pl
from jax.experimental.pallas import tpu as pltpu
from jax.experimental.pallas import tpu_sc as plsc

N = 10000
E = 320000
D = 128
H = 128
L = 3

NC = 2
NS = 16
NW = NC * NS
EPT = E // NW
CH = 80
NCH = EPT // CH
RPT = 624
NTAIL = N - NS * RPT

_mesh = plsc.VectorSubcoreMesh(core_axis_name="c", subcore_axis_name="s")


@functools.partial(
    pl.kernel,
    out_type=jax.ShapeDtypeStruct((NC, N, D), jnp.float32),
    mesh=_mesh,
    scratch_types=[
        pltpu.VMEM((NCH, CH), jnp.int32),
        pltpu.VMEM((NCH, CH), jnp.int32),
        pltpu.VMEM((CH, D), jnp.float32),
        pltpu.VMEM_SHARED((N, D), jnp.float32),
        pltpu.SemaphoreType.DMA,
    ],
)
def _segsum_sc(h_hbm, src_hbm, dst_hbm, zero_hbm, out_hbm,
               src_v, dst_v, rows_v, agg_sh, sem):
    c = lax.axis_index("c")
    s = lax.axis_index("s")
    w = c * NS + s
    pltpu.sync_copy(zero_hbm.at[pl.ds(s * RPT, RPT)],
                    agg_sh.at[pl.ds(s * RPT, RPT)])

    @pl.when(s == NS - 1)
    def _():
        pltpu.sync_copy(zero_hbm.at[pl.ds(NS * RPT, NTAIL)],
                        agg_sh.at[pl.ds(NS * RPT, NTAIL)])
    pltpu.sync_copy(src_hbm.at[w], src_v)
    pltpu.sync_copy(dst_hbm.at[w], dst_v)
    plsc.subcore_barrier()

    @pl.loop(0, NCH)
    def _(j):
        pltpu.async_copy(h_hbm.at[src_v.at[j]], rows_v, sem).wait()
        pltpu.sync_copy(rows_v, agg_sh.at[dst_v.at[j]], add=True)

    plsc.subcore_barrier()
    pltpu.sync_copy(agg_sh.at[pl.ds(s * RPT, RPT)],
                    out_hbm.at[c].at[pl.ds(s * RPT, RPT)])

    @pl.when(s == NS - 1)
    def _():
        pltpu.sync_copy(agg_sh.at[pl.ds(NS * RPT, NTAIL)],
                        out_hbm.at[c].at[pl.ds(NS * RPT, NTAIL)])


def _mm(a, b):
    return jnp.dot(a.astype(jnp.bfloat16), b.astype(jnp.bfloat16),
                   preferred_element_type=jnp.float32)


def _bn_relu(z, g, b):
    m = jnp.mean(z, axis=0, keepdims=True)
    zc = z - m
    v = jnp.mean(zc * zc, axis=0, keepdims=True)
    return jnp.maximum(zc / jnp.sqrt(v + 1e-5) * g + b, 0.0)


def _layer_body(h_ref, agg_ref, sc_ref, w1_ref, b1_ref, w2_ref, b2_ref,
                g_ref, be_ref, o_ref):
    z = h_ref[...] * sc_ref[...] + (agg_ref[0] + agg_ref[1])
    z = _mm(z, w1_ref[...]) + b1_ref[...]
    z = jnp.maximum(z, 0.0)
    z = _mm(z, w2_ref[...]) + b2_ref[...]
    o_ref[...] = _bn_relu(z, g_ref[...], be_ref[...])


_layer_call = pl.pallas_call(
    _layer_body, out_shape=jax.ShapeDtypeStruct((N, H), jnp.float32))


def _last_body(h_ref, agg_ref, sc_ref, w1_ref, b1_ref, w2_ref, b2_ref,
               g_ref, be_ref,
               wc0_ref, bc0_ref, gc0_ref, bec0_ref,
               wc1_ref, bc1_ref, gc1_ref, bec1_ref,
               wc2_ref, bc2_ref, o_ref):
    z = h_ref[...] * sc_ref[...] + (agg_ref[0] + agg_ref[1])
    z = _mm(z, w1_ref[...]) + b1_ref[...]
    z = jnp.maximum(z, 0.0)
    z = _mm(z, w2_ref[...]) + b2_ref[...]
    h = _bn_relu(z, g_ref[...], be_ref[...])
    h = _bn_relu(_mm(h, wc0_ref[...])
                 + bc0_ref[...], gc0_ref[...], bec0_ref[...])
    h = _bn_relu(_mm(h, wc1_ref[...])
                 + bc1_ref[...], gc1_ref[...], bec1_ref[...])
    o_ref[...] = (_mm(h, wc2_ref[...])
                  + bc2_ref[...])


_last_call = pl.pallas_call(
    _last_body, out_shape=jax.ShapeDtypeStruct((N, 3), jnp.float32))


def kernel(x, edge_index, params):
    src3 = edge_index[0].reshape(NW, NCH, CH)
    dst3 = edge_index[1].reshape(NW, NCH, CH)
    zeros = jnp.zeros((N, D), jnp.float32)
    h = x
    for i in range(L):
        agg2 = _segsum_sc(h, src3, dst3, zeros)
        sc = (1.0 + params['eps%d' % i]).reshape(1, 1)
        mlp = (sc, params['W1_%d' % i], params['b1_%d' % i].reshape(1, H),
               params['W2_%d' % i], params['b2_%d' % i].reshape(1, H),
               params['g%d' % i].reshape(1, H), params['be%d' % i].reshape(1, H))
        if i < L - 1:
            h = _layer_call(h, agg2, *mlp)
        else:
            logits = _last_call(
                h, agg2, *mlp,
                params['Wc0'], params['bc0'].reshape(1, H),
                params['gc0'].reshape(1, H), params['bec0'].reshape(1, H),
                params['Wc1'], params['bc1'].reshape(1, H),
                params['gc1'].reshape(1, H), params['bec1'].reshape(1, H),
                params['Wc2'], params['bc2'].reshape(1, 3))
    return logits
